# baseline (speedup 1.0000x reference)
import functools

import jax
import jax.numpy as jnp
from jax import lax
from jax.experimental import pallas as pl
from jax.experimental.pallas import tpu as pltpu
from jax.experimental.pallas import tpu_sc as plsc

GAMMA = 1e-10
K = 10
NUM_CORES = 2
NUM_SUBCORES = 16
LANES = 16
PHASES = 2
BLOCK_ROWS = 512


def _sc_thresh_kernel(phase_rows, row_len):
    nw = NUM_CORES * NUM_SUBCORES
    rows_per_w = phase_rows // nw
    n_chunks = rows_per_w // LANES
    chunk_words = LANES * row_len
    half = row_len // 2

    mesh = plsc.VectorSubcoreMesh(core_axis_name="c", subcore_axis_name="s")

    @functools.partial(
        pl.kernel,
        out_type=jax.ShapeDtypeStruct((nw, n_chunks, LANES), jnp.float32),
        mesh=mesh,
        compiler_params=pltpu.CompilerParams(needs_layout_passes=False),
        scratch_types=[
            pltpu.VMEM((chunk_words,), jnp.float32),
            pltpu.VMEM((chunk_words,), jnp.float32),
            pltpu.VMEM((n_chunks, LANES), jnp.float32),
            pltpu.SemaphoreType.DMA,
            pltpu.SemaphoreType.DMA,
        ],
    )
    def body(t_hbm, th_hbm, t0, t1, th_all, mt0, mt1):
        wid = lax.axis_index("s") * NUM_CORES + lax.axis_index("c")
        row0 = wid * rows_per_w
        lane = lax.iota(jnp.int32, LANES)
        base_a = lane * row_len
        base_b = base_a + half

        def start_in(g, t_buf, t_sem):
            off = (row0 + g * LANES) * row_len
            pltpu.make_async_copy(
                t_hbm.at[pl.ds(off, chunk_words)], t_buf, t_sem).start()

        def wait_in(g, t_buf, t_sem):
            off = (row0 + g * LANES) * row_len
            pltpu.make_async_copy(
                t_hbm.at[pl.ds(off, chunk_words)], t_buf, t_sem).wait()

        def compute(g, t_buf):
            neg_inf = jnp.full((LANES,), -jnp.inf, jnp.float32)

            @pl.loop(0, half, init_carry=(neg_inf,) * (2 * K), unroll=2)
            def p1(i, carry):
                ta = carry[:K]
                tb = carry[K:]
                xa = plsc.load_gather(t_buf, [base_a + i])
                xb = plsc.load_gather(t_buf, [base_b + i])
                na, nb = [], []
                for j in range(K):
                    na.append(jnp.maximum(ta[j], xa))
                    xa = jnp.minimum(ta[j], xa)
                    nb.append(jnp.maximum(tb[j], xb))
                    xb = jnp.minimum(tb[j], xb)
                return tuple(na) + tuple(nb)

            ta = p1[:K]
            tb = p1[K:]
            m = [jnp.maximum(ta[j], tb[K - 1 - j]) for j in range(K)]
            while len(m) > 1:
                m = [jnp.minimum(m[2 * i], m[2 * i + 1])
                     for i in range(len(m) // 2)] + m[len(m) & ~1:]
            th_all[g, :] = m[0]

        start_in(0, t0, mt0)

        @pl.loop(0, n_chunks // 2)
        def outer(p):
            g0 = 2 * p
            start_in(g0 + 1, t1, mt1)
            wait_in(g0, t0, mt0)
            compute(g0, t0)

            @pl.when(p < n_chunks // 2 - 1)
            def _():
                start_in(g0 + 2, t0, mt0)

            wait_in(g0 + 1, t1, mt1)
            compute(g0 + 1, t1)

        pltpu.sync_copy(th_all, th_hbm.at[wid])

    return body


def _tc_loss_phase(pred_s, pred_t, thresh, acc_in, n_rows, row_len,
                   phase_rows, phase, is_last):
    br = BLOCK_ROWS
    nb = phase_rows // br
    b0 = phase * nb

    def body(s_ref, t_ref, th_ref, a_ref, o_ref, acc_ref):
        b = pl.program_id(0)
        s = s_ref[...]
        t = t_ref[...]
        th = th_ref[...].reshape(br, 1)
        s_top = jnp.sum(jnp.where(t >= th, s, 0.0), axis=1, keepdims=True)
        s_all = jnp.sum(s, axis=1, keepdims=True)
        d = s_top * jnp.float32(1.0 / K) - (s_all - s_top) * jnp.float32(
            1.0 / (row_len - K)
        )
        sig = 1.0 / (1.0 + jnp.exp(-d))
        part = jnp.sum(jnp.log(jnp.float32(GAMMA) + sig))

        @pl.when(b == 0)
        def _():
            acc_ref[0] = a_ref[0, 0]

        acc_ref[0] += part

        @pl.when(b == nb - 1)
        def _():
            if is_last:
                o_ref[0, 0] = -acc_ref[0] / n_rows
            else:
                o_ref[0, 0] = acc_ref[0]

    return pl.pallas_call(
        body,
        grid=(nb,),
        in_specs=[
            pl.BlockSpec((br, row_len), lambda b, b0=b0: (b0 + b, 0)),
            pl.BlockSpec((br, row_len), lambda b, b0=b0: (b0 + b, 0)),
            pl.BlockSpec((1, 1, br), lambda b: (b, 0, 0)),
            pl.BlockSpec(memory_space=pltpu.SMEM),
        ],
        out_specs=pl.BlockSpec(memory_space=pltpu.SMEM),
        out_shape=jax.ShapeDtypeStruct((1, 1), jnp.float32),
        scratch_shapes=[pltpu.SMEM((1,), jnp.float32)],
    )(pred_s, pred_t, thresh, acc_in)


def kernel(pred_s, pred_t, k, list_len):
    n_rows, row_len = pred_s.shape
    phase_rows = n_rows // PHASES

    sc = _sc_thresh_kernel(phase_rows, row_len)
    acc = jnp.zeros((1, 1), jnp.float32)
    for p in range(PHASES):
        t_flat = pred_t[p * phase_rows:(p + 1) * phase_rows].reshape(-1)
        th = sc(t_flat).reshape(phase_rows // BLOCK_ROWS, 1, BLOCK_ROWS)
        acc = _tc_loss_phase(pred_s, pred_t, th, acc, n_rows, row_len,
                             phase_rows, p, p == PHASES - 1)
    return acc[0, 0]


# speedup vs baseline: 1.0120x; 1.0120x over previous
import functools

import jax
import jax.numpy as jnp
from jax import lax
from jax.experimental import pallas as pl
from jax.experimental.pallas import tpu as pltpu
from jax.experimental.pallas import tpu_sc as plsc

GAMMA = 1e-10
K = 10
NUM_CORES = 2
NUM_SUBCORES = 16
LANES = 16
PHASES = 2
BLOCK_ROWS = 512


def _sc_thresh_kernel(phase_rows, row_len):
    nw = NUM_CORES * NUM_SUBCORES
    rows_per_w = phase_rows // nw
    n_chunks = rows_per_w // LANES
    half = row_len // 2

    mesh = plsc.VectorSubcoreMesh(core_axis_name="c", subcore_axis_name="s")

    @functools.partial(
        pl.kernel,
        out_type=jax.ShapeDtypeStruct((nw, n_chunks, LANES), jnp.float32),
        mesh=mesh,
        compiler_params=pltpu.CompilerParams(needs_layout_passes=False),
        scratch_types=[
            pltpu.VMEM((LANES, row_len), jnp.float32),
            pltpu.VMEM((LANES, row_len), jnp.float32),
            pltpu.VMEM((n_chunks, LANES), jnp.float32),
            pltpu.SemaphoreType.DMA,
            pltpu.SemaphoreType.DMA,
        ],
    )
    def body(t_hbm, th_hbm, t0, t1, th_all, mt0, mt1):
        wid = lax.axis_index("s") * NUM_CORES + lax.axis_index("c")
        row0 = wid * rows_per_w
        lane = lax.iota(jnp.int32, LANES)

        def start_in(g, t_buf, t_sem):
            r = row0 + g * LANES
            pltpu.make_async_copy(
                t_hbm.at[pl.ds(r, LANES), :], t_buf, t_sem).start()

        def wait_in(g, t_buf, t_sem):
            r = row0 + g * LANES
            pltpu.make_async_copy(
                t_hbm.at[pl.ds(r, LANES), :], t_buf, t_sem).wait()

        def compute(g, t_buf):
            neg_inf = jnp.full((LANES,), -jnp.inf, jnp.float32)

            @pl.loop(0, half, init_carry=(neg_inf,) * (2 * K), unroll=2)
            def p1(i, carry):
                ta = carry[:K]
                tb = carry[K:]
                ia = jnp.full((LANES,), 0, jnp.int32) + i
                xa = plsc.load_gather(t_buf, [lane, ia])
                xb = plsc.load_gather(t_buf, [lane, ia + half])
                na, nb = [], []
                for j in range(K):
                    na.append(jnp.maximum(ta[j], xa))
                    xa = jnp.minimum(ta[j], xa)
                    nb.append(jnp.maximum(tb[j], xb))
                    xb = jnp.minimum(tb[j], xb)
                return tuple(na) + tuple(nb)

            ta = p1[:K]
            tb = p1[K:]
            m = [jnp.maximum(ta[j], tb[K - 1 - j]) for j in range(K)]
            while len(m) > 1:
                m = [jnp.minimum(m[2 * i], m[2 * i + 1])
                     for i in range(len(m) // 2)] + m[len(m) & ~1:]
            th_all[g, :] = m[0]

        start_in(0, t0, mt0)

        @pl.loop(0, n_chunks // 2)
        def outer(p):
            g0 = 2 * p
            start_in(g0 + 1, t1, mt1)
            wait_in(g0, t0, mt0)
            compute(g0, t0)

            @pl.when(p < n_chunks // 2 - 1)
            def _():
                start_in(g0 + 2, t0, mt0)

            wait_in(g0 + 1, t1, mt1)
            compute(g0 + 1, t1)

        pltpu.sync_copy(th_all, th_hbm.at[wid])

    return body


def _tc_loss_phase(pred_s, pred_t, thresh, acc_in, n_rows, row_len,
                   phase_rows, phase, is_last):
    br = BLOCK_ROWS
    nb = phase_rows // br
    b0 = phase * nb

    def body(s_ref, t_ref, th_ref, a_ref, o_ref, acc_ref):
        b = pl.program_id(0)
        s = s_ref[...]
        t = t_ref[...]
        th = th_ref[...].reshape(br, 1)
        s_top = jnp.sum(jnp.where(t >= th, s, 0.0), axis=1, keepdims=True)
        s_all = jnp.sum(s, axis=1, keepdims=True)
        d = s_top * jnp.float32(1.0 / K) - (s_all - s_top) * jnp.float32(
            1.0 / (row_len - K)
        )
        sig = 1.0 / (1.0 + jnp.exp(-d))
        part = jnp.sum(jnp.log(jnp.float32(GAMMA) + sig))

        @pl.when(b == 0)
        def _():
            acc_ref[0] = a_ref[0, 0]

        acc_ref[0] += part

        @pl.when(b == nb - 1)
        def _():
            if is_last:
                o_ref[0, 0] = -acc_ref[0] / n_rows
            else:
                o_ref[0, 0] = acc_ref[0]

    return pl.pallas_call(
        body,
        grid=(nb,),
        in_specs=[
            pl.BlockSpec((br, row_len), lambda b, b0=b0: (b0 + b, 0)),
            pl.BlockSpec((br, row_len), lambda b, b0=b0: (b0 + b, 0)),
            pl.BlockSpec((1, 1, br), lambda b: (b, 0, 0)),
            pl.BlockSpec(memory_space=pltpu.SMEM),
        ],
        out_specs=pl.BlockSpec(memory_space=pltpu.SMEM),
        out_shape=jax.ShapeDtypeStruct((1, 1), jnp.float32),
        scratch_shapes=[pltpu.SMEM((1,), jnp.float32)],
    )(pred_s, pred_t, thresh, acc_in)


def kernel(pred_s, pred_t, k, list_len):
    n_rows, row_len = pred_s.shape
    phase_rows = n_rows // PHASES

    sc = _sc_thresh_kernel(phase_rows, row_len)
    acc = jnp.zeros((1, 1), jnp.float32)
    for p in range(PHASES):
        t_slice = lax.slice_in_dim(pred_t, p * phase_rows,
                                   (p + 1) * phase_rows, axis=0)
        th = sc(t_slice).reshape(phase_rows // BLOCK_ROWS, 1, BLOCK_ROWS)
        acc = _tc_loss_phase(pred_s, pred_t, th, acc, n_rows, row_len,
                             phase_rows, p, p == PHASES - 1)
    return acc[0, 0]


# speedup vs baseline: 1.1001x; 1.0871x over previous
import functools

import jax
import jax.numpy as jnp
from jax import lax
from jax.experimental import pallas as pl
from jax.experimental.pallas import tpu as pltpu
from jax.experimental.pallas import tpu_sc as plsc

GAMMA = 1e-10
K = 10
NUM_CORES = 2
NUM_SUBCORES = 16
LANES = 16
PHASES = 2
BLOCK_ROWS = 512


def _sc_thresh_kernel(phase_rows, row_len, phase):
    nw = NUM_CORES * NUM_SUBCORES
    rows_per_w = phase_rows // nw
    n_chunks = rows_per_w // LANES
    half = row_len // 2

    mesh = plsc.VectorSubcoreMesh(core_axis_name="c", subcore_axis_name="s")

    @functools.partial(
        pl.kernel,
        out_type=jax.ShapeDtypeStruct((nw, n_chunks, LANES), jnp.float32),
        mesh=mesh,
        compiler_params=pltpu.CompilerParams(needs_layout_passes=False),
        scratch_types=[
            pltpu.VMEM((LANES, row_len), jnp.float32),
            pltpu.VMEM((LANES, row_len), jnp.float32),
            pltpu.VMEM((n_chunks, LANES), jnp.float32),
            pltpu.SemaphoreType.DMA,
            pltpu.SemaphoreType.DMA,
        ],
    )
    def body(t_hbm, th_hbm, t0, t1, th_all, mt0, mt1):
        wid = lax.axis_index("s") * NUM_CORES + lax.axis_index("c")
        row0 = phase * phase_rows + wid * rows_per_w
        lane = lax.iota(jnp.int32, LANES)

        def start_in(g, t_buf, t_sem):
            r = row0 + g * LANES
            pltpu.make_async_copy(
                t_hbm.at[pl.ds(r, LANES), :], t_buf, t_sem).start()

        def wait_in(g, t_buf, t_sem):
            r = row0 + g * LANES
            pltpu.make_async_copy(
                t_hbm.at[pl.ds(r, LANES), :], t_buf, t_sem).wait()

        def compute(g, t_buf):
            neg_inf = jnp.full((LANES,), -jnp.inf, jnp.float32)

            @pl.loop(0, half, init_carry=(neg_inf,) * (2 * K), unroll=2)
            def p1(i, carry):
                ta = carry[:K]
                tb = carry[K:]
                ia = jnp.full((LANES,), 0, jnp.int32) + i
                xa = plsc.load_gather(t_buf, [lane, ia])
                xb = plsc.load_gather(t_buf, [lane, ia + half])
                na, nb = [], []
                for j in range(K):
                    na.append(jnp.maximum(ta[j], xa))
                    xa = jnp.minimum(ta[j], xa)
                    nb.append(jnp.maximum(tb[j], xb))
                    xb = jnp.minimum(tb[j], xb)
                return tuple(na) + tuple(nb)

            ta = p1[:K]
            tb = p1[K:]
            m = [jnp.maximum(ta[j], tb[K - 1 - j]) for j in range(K)]
            while len(m) > 1:
                m = [jnp.minimum(m[2 * i], m[2 * i + 1])
                     for i in range(len(m) // 2)] + m[len(m) & ~1:]
            th_all[g, :] = m[0]

        start_in(0, t0, mt0)

        @pl.loop(0, n_chunks // 2)
        def outer(p):
            g0 = 2 * p
            start_in(g0 + 1, t1, mt1)
            wait_in(g0, t0, mt0)
            compute(g0, t0)

            @pl.when(p < n_chunks // 2 - 1)
            def _():
                start_in(g0 + 2, t0, mt0)

            wait_in(g0 + 1, t1, mt1)
            compute(g0 + 1, t1)

        pltpu.sync_copy(th_all, th_hbm.at[wid])

    return body


def _tc_loss_phase(pred_s, pred_t, thresh, acc_in, n_rows, row_len,
                   phase_rows, phase, is_last):
    br = BLOCK_ROWS
    nb = phase_rows // br
    b0 = phase * nb

    def body(s_ref, t_ref, th_ref, a_ref, o_ref, acc_ref):
        b = pl.program_id(0)
        s = s_ref[...]
        t = t_ref[...]
        th = th_ref[...].reshape(br, 1)
        s_top = jnp.sum(jnp.where(t >= th, s, 0.0), axis=1, keepdims=True)
        s_all = jnp.sum(s, axis=1, keepdims=True)
        d = s_top * jnp.float32(1.0 / K) - (s_all - s_top) * jnp.float32(
            1.0 / (row_len - K)
        )
        sig = 1.0 / (1.0 + jnp.exp(-d))
        part = jnp.sum(jnp.log(jnp.float32(GAMMA) + sig))

        @pl.when(b == 0)
        def _():
            acc_ref[0] = a_ref[0, 0]

        acc_ref[0] += part

        @pl.when(b == nb - 1)
        def _():
            if is_last:
                o_ref[0, 0] = -acc_ref[0] / n_rows
            else:
                o_ref[0, 0] = acc_ref[0]

    return pl.pallas_call(
        body,
        grid=(nb,),
        in_specs=[
            pl.BlockSpec((br, row_len), lambda b, b0=b0: (b0 + b, 0)),
            pl.BlockSpec((br, row_len), lambda b, b0=b0: (b0 + b, 0)),
            pl.BlockSpec((1, 1, br), lambda b: (b, 0, 0)),
            pl.BlockSpec(memory_space=pltpu.SMEM),
        ],
        out_specs=pl.BlockSpec(memory_space=pltpu.SMEM),
        out_shape=jax.ShapeDtypeStruct((1, 1), jnp.float32),
        scratch_shapes=[pltpu.SMEM((1,), jnp.float32)],
    )(pred_s, pred_t, thresh, acc_in)


def kernel(pred_s, pred_t, k, list_len):
    n_rows, row_len = pred_s.shape
    phase_rows = n_rows // PHASES

    acc = jnp.zeros((1, 1), jnp.float32)
    for p in range(PHASES):
        sc = _sc_thresh_kernel(phase_rows, row_len, p)
        th = sc(pred_t).reshape(phase_rows // BLOCK_ROWS, 1, BLOCK_ROWS)
        acc = _tc_loss_phase(pred_s, pred_t, th, acc, n_rows, row_len,
                             phase_rows, p, p == PHASES - 1)
    return acc[0, 0]


# speedup vs baseline: 1.1357x; 1.0323x over previous
import functools

import jax
import jax.numpy as jnp
from jax import lax
from jax.experimental import pallas as pl
from jax.experimental.pallas import tpu as pltpu
from jax.experimental.pallas import tpu_sc as plsc

GAMMA = 1e-10
K = 10
NUM_CORES = 2
NUM_SUBCORES = 16
LANES = 16
PHASE_ROWS = (10240, 6144)
BLOCK_ROWS = 512


def _sc_thresh_kernel(phase_rows, row_len, row_offset):
    nw = NUM_CORES * NUM_SUBCORES
    rows_per_w = phase_rows // nw
    n_chunks = rows_per_w // LANES
    half = row_len // 2

    mesh = plsc.VectorSubcoreMesh(core_axis_name="c", subcore_axis_name="s")

    @functools.partial(
        pl.kernel,
        out_type=jax.ShapeDtypeStruct((nw, n_chunks, LANES), jnp.float32),
        mesh=mesh,
        compiler_params=pltpu.CompilerParams(needs_layout_passes=False),
        scratch_types=[
            pltpu.VMEM((LANES, row_len), jnp.float32),
            pltpu.VMEM((LANES, row_len), jnp.float32),
            pltpu.VMEM((n_chunks, LANES), jnp.float32),
            pltpu.SemaphoreType.DMA,
            pltpu.SemaphoreType.DMA,
        ],
    )
    def body(t_hbm, th_hbm, t0, t1, th_all, mt0, mt1):
        wid = lax.axis_index("s") * NUM_CORES + lax.axis_index("c")
        row0 = row_offset + wid * rows_per_w
        lane = lax.iota(jnp.int32, LANES)

        def start_in(g, t_buf, t_sem):
            r = row0 + g * LANES
            pltpu.make_async_copy(
                t_hbm.at[pl.ds(r, LANES), :], t_buf, t_sem).start()

        def wait_in(g, t_buf, t_sem):
            r = row0 + g * LANES
            pltpu.make_async_copy(
                t_hbm.at[pl.ds(r, LANES), :], t_buf, t_sem).wait()

        def compute(g, t_buf):
            neg_inf = jnp.full((LANES,), -jnp.inf, jnp.float32)

            @pl.loop(0, half, init_carry=(neg_inf,) * (2 * K), unroll=2)
            def p1(i, carry):
                ta = carry[:K]
                tb = carry[K:]
                ia = jnp.full((LANES,), 0, jnp.int32) + i
                xa = plsc.load_gather(t_buf, [lane, ia])
                xb = plsc.load_gather(t_buf, [lane, ia + half])
                na, nb = [], []
                for j in range(K):
                    na.append(jnp.maximum(ta[j], xa))
                    xa = jnp.minimum(ta[j], xa)
                    nb.append(jnp.maximum(tb[j], xb))
                    xb = jnp.minimum(tb[j], xb)
                return tuple(na) + tuple(nb)

            ta = p1[:K]
            tb = p1[K:]
            m = [jnp.maximum(ta[j], tb[K - 1 - j]) for j in range(K)]
            while len(m) > 1:
                m = [jnp.minimum(m[2 * i], m[2 * i + 1])
                     for i in range(len(m) // 2)] + m[len(m) & ~1:]
            th_all[g, :] = m[0]

        start_in(0, t0, mt0)

        @pl.loop(0, n_chunks // 2)
        def outer(p):
            g0 = 2 * p
            start_in(g0 + 1, t1, mt1)
            wait_in(g0, t0, mt0)
            compute(g0, t0)

            @pl.when(p < n_chunks // 2 - 1)
            def _():
                start_in(g0 + 2, t0, mt0)

            wait_in(g0 + 1, t1, mt1)
            compute(g0 + 1, t1)

        pltpu.sync_copy(th_all, th_hbm.at[wid])

    return body


def _tc_loss_phase(pred_s, pred_t, thresh, acc_in, n_rows, row_len,
                   phase_rows, row_offset, is_last):
    br = BLOCK_ROWS
    nb = phase_rows // br
    b0 = row_offset // br

    def body(s_ref, t_ref, th_ref, a_ref, o_ref, acc_ref):
        b = pl.program_id(0)
        s = s_ref[...]
        t = t_ref[...]
        th = th_ref[...].reshape(br, 1)
        s_top = jnp.sum(jnp.where(t >= th, s, 0.0), axis=1, keepdims=True)
        s_all = jnp.sum(s, axis=1, keepdims=True)
        d = s_top * jnp.float32(1.0 / K) - (s_all - s_top) * jnp.float32(
            1.0 / (row_len - K)
        )
        sig = 1.0 / (1.0 + jnp.exp(-d))
        part = jnp.sum(jnp.log(jnp.float32(GAMMA) + sig))

        @pl.when(b == 0)
        def _():
            acc_ref[0] = a_ref[0, 0]

        acc_ref[0] += part

        @pl.when(b == nb - 1)
        def _():
            if is_last:
                o_ref[0, 0] = -acc_ref[0] / n_rows
            else:
                o_ref[0, 0] = acc_ref[0]

    return pl.pallas_call(
        body,
        grid=(nb,),
        in_specs=[
            pl.BlockSpec((br, row_len), lambda b, b0=b0: (b0 + b, 0)),
            pl.BlockSpec((br, row_len), lambda b, b0=b0: (b0 + b, 0)),
            pl.BlockSpec((1, 1, br), lambda b: (b, 0, 0)),
            pl.BlockSpec(memory_space=pltpu.SMEM),
        ],
        out_specs=pl.BlockSpec(memory_space=pltpu.SMEM),
        out_shape=jax.ShapeDtypeStruct((1, 1), jnp.float32),
        scratch_shapes=[pltpu.SMEM((1,), jnp.float32)],
    )(pred_s, pred_t, thresh, acc_in)


def kernel(pred_s, pred_t, k, list_len):
    n_rows, row_len = pred_s.shape
    assert sum(PHASE_ROWS) == n_rows

    acc = jnp.zeros((1, 1), jnp.float32)
    row_offset = 0
    for p, phase_rows in enumerate(PHASE_ROWS):
        sc = _sc_thresh_kernel(phase_rows, row_len, row_offset)
        th = sc(pred_t).reshape(phase_rows // BLOCK_ROWS, 1, BLOCK_ROWS)
        acc = _tc_loss_phase(pred_s, pred_t, th, acc, n_rows, row_len,
                             phase_rows, row_offset,
                             p == len(PHASE_ROWS) - 1)
        row_offset += phase_rows
    return acc[0, 0]


# speedup vs baseline: 1.1474x; 1.0103x over previous
import functools

import jax
import jax.numpy as jnp
from jax import lax
from jax.experimental import pallas as pl
from jax.experimental.pallas import tpu as pltpu
from jax.experimental.pallas import tpu_sc as plsc

GAMMA = 1e-10
K = 10
NUM_CORES = 2
NUM_SUBCORES = 16
LANES = 16
PHASE_ROWS = (8192, 5120, 3072)
BLOCK_ROWS = 512


def _sc_thresh_kernel(phase_rows, row_len, row_offset):
    nw = NUM_CORES * NUM_SUBCORES
    rows_per_w = phase_rows // nw
    n_chunks = rows_per_w // LANES
    half = row_len // 2

    mesh = plsc.VectorSubcoreMesh(core_axis_name="c", subcore_axis_name="s")

    @functools.partial(
        pl.kernel,
        out_type=jax.ShapeDtypeStruct((nw, n_chunks, LANES), jnp.float32),
        mesh=mesh,
        compiler_params=pltpu.CompilerParams(needs_layout_passes=False),
        scratch_types=[
            pltpu.VMEM((LANES, row_len), jnp.float32),
            pltpu.VMEM((LANES, row_len), jnp.float32),
            pltpu.VMEM((n_chunks, LANES), jnp.float32),
            pltpu.SemaphoreType.DMA,
            pltpu.SemaphoreType.DMA,
        ],
    )
    def body(t_hbm, th_hbm, t0, t1, th_all, mt0, mt1):
        wid = lax.axis_index("s") * NUM_CORES + lax.axis_index("c")
        row0 = row_offset + wid * rows_per_w
        lane = lax.iota(jnp.int32, LANES)

        def start_in(g, t_buf, t_sem):
            r = row0 + g * LANES
            pltpu.make_async_copy(
                t_hbm.at[pl.ds(r, LANES), :], t_buf, t_sem).start()

        def wait_in(g, t_buf, t_sem):
            r = row0 + g * LANES
            pltpu.make_async_copy(
                t_hbm.at[pl.ds(r, LANES), :], t_buf, t_sem).wait()

        def compute(g, t_buf):
            neg_inf = jnp.full((LANES,), -jnp.inf, jnp.float32)

            @pl.loop(0, half, init_carry=(neg_inf,) * (2 * K), unroll=2)
            def p1(i, carry):
                ta = carry[:K]
                tb = carry[K:]
                ia = jnp.full((LANES,), 0, jnp.int32) + i
                xa = plsc.load_gather(t_buf, [lane, ia])
                xb = plsc.load_gather(t_buf, [lane, ia + half])
                na, nb = [], []
                for j in range(K):
                    na.append(jnp.maximum(ta[j], xa))
                    xa = jnp.minimum(ta[j], xa)
                    nb.append(jnp.maximum(tb[j], xb))
                    xb = jnp.minimum(tb[j], xb)
                return tuple(na) + tuple(nb)

            ta = p1[:K]
            tb = p1[K:]
            m = [jnp.maximum(ta[j], tb[K - 1 - j]) for j in range(K)]
            while len(m) > 1:
                m = [jnp.minimum(m[2 * i], m[2 * i + 1])
                     for i in range(len(m) // 2)] + m[len(m) & ~1:]
            th_all[g, :] = m[0]

        start_in(0, t0, mt0)

        @pl.loop(0, n_chunks // 2)
        def outer(p):
            g0 = 2 * p
            start_in(g0 + 1, t1, mt1)
            wait_in(g0, t0, mt0)
            compute(g0, t0)

            @pl.when(p < n_chunks // 2 - 1)
            def _():
                start_in(g0 + 2, t0, mt0)

            wait_in(g0 + 1, t1, mt1)
            compute(g0 + 1, t1)

        pltpu.sync_copy(th_all, th_hbm.at[wid])

    return body


def _tc_loss_phase(pred_s, pred_t, thresh, acc_in, n_rows, row_len,
                   phase_rows, row_offset, is_last):
    br = BLOCK_ROWS
    nb = phase_rows // br
    b0 = row_offset // br

    def body(s_ref, t_ref, th_ref, a_ref, o_ref, acc_ref):
        b = pl.program_id(0)
        s = s_ref[...]
        t = t_ref[...]
        th = th_ref[...].reshape(br, 1)
        s_top = jnp.sum(jnp.where(t >= th, s, 0.0), axis=1, keepdims=True)
        s_all = jnp.sum(s, axis=1, keepdims=True)
        d = s_top * jnp.float32(1.0 / K) - (s_all - s_top) * jnp.float32(
            1.0 / (row_len - K)
        )
        sig = 1.0 / (1.0 + jnp.exp(-d))
        part = jnp.sum(jnp.log(jnp.float32(GAMMA) + sig))

        @pl.when(b == 0)
        def _():
            acc_ref[0] = a_ref[0, 0]

        acc_ref[0] += part

        @pl.when(b == nb - 1)
        def _():
            if is_last:
                o_ref[0, 0] = -acc_ref[0] / n_rows
            else:
                o_ref[0, 0] = acc_ref[0]

    return pl.pallas_call(
        body,
        grid=(nb,),
        in_specs=[
            pl.BlockSpec((br, row_len), lambda b, b0=b0: (b0 + b, 0)),
            pl.BlockSpec((br, row_len), lambda b, b0=b0: (b0 + b, 0)),
            pl.BlockSpec((1, 1, br), lambda b: (b, 0, 0)),
            pl.BlockSpec(memory_space=pltpu.SMEM),
        ],
        out_specs=pl.BlockSpec(memory_space=pltpu.SMEM),
        out_shape=jax.ShapeDtypeStruct((1, 1), jnp.float32),
        scratch_shapes=[pltpu.SMEM((1,), jnp.float32)],
    )(pred_s, pred_t, thresh, acc_in)


def kernel(pred_s, pred_t, k, list_len):
    n_rows, row_len = pred_s.shape
    assert sum(PHASE_ROWS) == n_rows

    acc = jnp.zeros((1, 1), jnp.float32)
    row_offset = 0
    for p, phase_rows in enumerate(PHASE_ROWS):
        sc = _sc_thresh_kernel(phase_rows, row_len, row_offset)
        th = sc(pred_t).reshape(phase_rows // BLOCK_ROWS, 1, BLOCK_ROWS)
        acc = _tc_loss_phase(pred_s, pred_t, th, acc, n_rows, row_len,
                             phase_rows, row_offset,
                             p == len(PHASE_ROWS) - 1)
        row_offset += phase_rows
    return acc[0, 0]
